# baseline (speedup 1.0000x reference)
import functools

import jax
import jax.numpy as jnp
from jax import lax
from jax.experimental import pallas as pl
from jax.experimental.pallas import tpu as pltpu
from jax.experimental.pallas import tpu_sc as plsc

_NE = 8
_TOPK = 2
_BI = 1024
_HB = _BI // 2
_LANES = 16


def _gate_body(x_ref, gw_ref, lg_ref):
    logits = jax.lax.dot_general(
        gw_ref[...], x_ref[...], (((1,), (1,)), ((), ())),
        preferred_element_type=jnp.float32)
    lg_ref[...] = 30.0 * jnp.tanh(logits * (1.0 / 30.0))


def _gate_logits(x, gate_w):
    t, hd = x.shape
    ne = gate_w.shape[0]
    return pl.pallas_call(
        _gate_body,
        out_shape=jax.ShapeDtypeStruct((ne, t), jnp.float32),
    )(x, gate_w)


def _sc_router_body(lg_hbm, cw_hbm, lg_v, cw_v, sem):
    wid = lax.axis_index("s") * 2 + lax.axis_index("c")
    nchunks = 128 // _LANES

    @pl.when(wid < nchunks)
    def _():
        base = wid * _LANES
        for e in range(_NE):
            pltpu.sync_copy(lg_hbm.at[e, pl.ds(base, _LANES)],
                            lg_v.at[e, :])
        lg = [lg_v[e, :] for e in range(_NE)]
        m = lg[0]
        for e in range(1, _NE):
            m = jnp.maximum(m, lg[e])
        ex = [jnp.exp(lg[e] - m) for e in range(_NE)]
        s = ex[0]
        for e in range(1, _NE):
            s = s + ex[e]
        p = [ex[e] / s for e in range(_NE)]
        zero = jnp.zeros((_LANES,), jnp.int32)
        one = jnp.ones((_LANES,), jnp.int32)
        idx0 = lax.iota(jnp.int32, _LANES)
        for e in range(_NE):
            rank = zero
            for j in range(_NE):
                if j == e:
                    continue
                gt = jnp.where(p[j] > p[e], one, zero)
                if j < e:
                    gt = jnp.where(p[j] == p[e], one, gt)
                rank = rank + gt
            cw_e = jnp.where(rank < _TOPK, p[e], 0.0)
            plsc.store_scatter(cw_v, [idx0 * _NE + e], cw_e)
        pltpu.sync_copy(cw_v, cw_hbm.at[pl.ds(base * _NE, _LANES * _NE)])


def _sc_router(logits_t):
    mesh = plsc.VectorSubcoreMesh(core_axis_name="c", subcore_axis_name="s")
    t = logits_t.shape[1]
    run = pl.kernel(
        _sc_router_body,
        mesh=mesh,
        compiler_params=pltpu.CompilerParams(needs_layout_passes=False),
        cost_estimate=pl.CostEstimate(
            flops=200_000_000, transcendentals=1_000_000,
            bytes_accessed=8_192),
        out_type=jax.ShapeDtypeStruct((t * _NE,), jnp.float32),
        scratch_types=[
            pltpu.VMEM((_NE, _LANES), jnp.float32),
            pltpu.VMEM((_LANES * _NE,), jnp.float32),
            pltpu.SemaphoreType.DMA,
        ],
    )
    return run(logits_t).reshape(t, _NE)


def _moe_body(x_ref, w1a_ref, w1b_ref, w3a_ref, w3b_ref,
              w2a_ref, w2b_ref, y_ref):
    i = pl.program_id(1)

    x = x_ref[...].astype(jnp.bfloat16)
    cd = (((1,), (1,)), ((), ()))
    ha = jax.lax.dot_general(x, w1a_ref[0].astype(jnp.bfloat16), cd,
                             preferred_element_type=jnp.float32)
    ga = jax.lax.dot_general(x, w3a_ref[0].astype(jnp.bfloat16), cd,
                             preferred_element_type=jnp.float32)
    acta = (ha * (1.0 / (1.0 + jnp.exp(-ha))) * ga).astype(jnp.bfloat16)
    hb = jax.lax.dot_general(x, w1b_ref[0].astype(jnp.bfloat16), cd,
                             preferred_element_type=jnp.float32)
    gb = jax.lax.dot_general(x, w3b_ref[0].astype(jnp.bfloat16), cd,
                             preferred_element_type=jnp.float32)
    actb = (hb * (1.0 / (1.0 + jnp.exp(-hb))) * gb).astype(jnp.bfloat16)
    part = (jax.lax.dot_general(acta, w2a_ref[0].astype(jnp.bfloat16), cd,
                                preferred_element_type=jnp.float32)
            + jax.lax.dot_general(actb, w2b_ref[0].astype(jnp.bfloat16), cd,
                                  preferred_element_type=jnp.float32))

    @pl.when(i == 0)
    def _():
        y_ref[0] = part

    @pl.when(i != 0)
    def _():
        y_ref[0] += part


def _experts(x, w1, w3, w2):
    t, hd = x.shape
    ne, di, _ = w1.shape
    ni = di // _BI
    up_a = pl.BlockSpec((1, _HB, hd), lambda e, i: (e, 2 * i, 0))
    up_b = pl.BlockSpec((1, _HB, hd), lambda e, i: (e, 2 * i + 1, 0))
    dn_a = pl.BlockSpec((1, hd, _HB), lambda e, i: (e, 0, 2 * i))
    dn_b = pl.BlockSpec((1, hd, _HB), lambda e, i: (e, 0, 2 * i + 1))
    return pl.pallas_call(
        _moe_body,
        grid=(ne, ni),
        in_specs=[
            pl.BlockSpec((t, hd), lambda e, i: (0, 0)),
            up_a, up_b, up_a, up_b, dn_a, dn_b,
        ],
        out_specs=pl.BlockSpec((1, t, hd), lambda e, i: (e, 0, 0)),
        out_shape=jax.ShapeDtypeStruct((ne, t, hd), jnp.float32),
    )(x, w1, w1, w3, w3, w2, w2)


def _combine_body(y_ref, cw_ref, out_ref):
    acc = cw_ref[:, 0:1] * y_ref[0]
    for e in range(1, _NE):
        acc += cw_ref[:, e:e + 1] * y_ref[e]
    out_ref[...] = acc


def _combine(y, cw):
    ne, t, hd = y.shape
    return pl.pallas_call(
        _combine_body,
        out_shape=jax.ShapeDtypeStruct((t, hd), jnp.float32),
    )(y, cw)


def kernel(hidden_states, gate_w, w1, w3, w2):
    orig_shape = hidden_states.shape
    x = hidden_states.reshape(-1, orig_shape[-1])
    y = _experts(x, w1, w3, w2)
    logits_t = _gate_logits(x, gate_w)
    cw = _sc_router(logits_t)
    out = _combine(y, cw)
    return out.reshape(orig_shape)


# speedup vs baseline: 1.2190x; 1.2190x over previous
import jax
import jax.numpy as jnp
from jax.experimental import pallas as pl
from jax.experimental.pallas import tpu as pltpu

_NE = 8
_TOPK = 2
_BI = 2048
_HB = _BI // 2


def _router(x, gw):
    logits = jax.lax.dot_general(
        x, gw, (((1,), (1,)), ((), ())), preferred_element_type=jnp.float32)
    logits = 30.0 * jnp.tanh(logits / 30.0)
    m = jnp.max(logits, axis=1, keepdims=True)
    ex = jnp.exp(logits - m)
    p = ex / jnp.sum(ex, axis=1, keepdims=True)
    cols = jax.lax.broadcasted_iota(jnp.int32, (1, _NE), 1)
    rank_cols = []
    for ee in range(_NE):
        pe = p[:, ee:ee + 1]
        gt = jnp.sum((p > pe).astype(jnp.int32), axis=1, keepdims=True)
        eq = jnp.sum(jnp.logical_and(p == pe, cols < ee).astype(jnp.int32),
                     axis=1, keepdims=True)
        rank_cols.append(gt + eq)
    rank = jnp.concatenate(rank_cols, axis=1)
    return jnp.where(rank < _TOPK, p, 0.0)


def _moe_body(x_ref, gw_ref, w1a_ref, w1b_ref, w3a_ref, w3b_ref,
              w2a_ref, w2b_ref, out_ref, cw_ref):
    e = pl.program_id(0)
    i = pl.program_id(1)
    first = jnp.logical_and(e == 0, i == 0)

    @pl.when(first)
    def _():
        cw_ref[...] = _router(x_ref[...], gw_ref[...])

    x = x_ref[...].astype(jnp.bfloat16)
    cd = (((1,), (1,)), ((), ()))
    ha = jax.lax.dot_general(x, w1a_ref[0].astype(jnp.bfloat16), cd,
                             preferred_element_type=jnp.float32)
    ga = jax.lax.dot_general(x, w3a_ref[0].astype(jnp.bfloat16), cd,
                             preferred_element_type=jnp.float32)
    acta = (ha * (1.0 / (1.0 + jnp.exp(-ha))) * ga).astype(jnp.bfloat16)
    hb = jax.lax.dot_general(x, w1b_ref[0].astype(jnp.bfloat16), cd,
                             preferred_element_type=jnp.float32)
    gb = jax.lax.dot_general(x, w3b_ref[0].astype(jnp.bfloat16), cd,
                             preferred_element_type=jnp.float32)
    actb = (hb * (1.0 / (1.0 + jnp.exp(-hb))) * gb).astype(jnp.bfloat16)
    part = (jax.lax.dot_general(acta, w2a_ref[0].astype(jnp.bfloat16), cd,
                                preferred_element_type=jnp.float32)
            + jax.lax.dot_general(actb, w2b_ref[0].astype(jnp.bfloat16), cd,
                                  preferred_element_type=jnp.float32))
    onehot = (jax.lax.broadcasted_iota(jnp.int32, (1, _NE), 1) == e
              ).astype(jnp.float32)
    cw_col = jnp.sum(cw_ref[...] * onehot, axis=1, keepdims=True)
    contrib = cw_col * part

    @pl.when(first)
    def _():
        out_ref[...] = contrib

    @pl.when(jnp.logical_not(first))
    def _():
        out_ref[...] += contrib


def kernel(hidden_states, gate_w, w1, w3, w2):
    orig_shape = hidden_states.shape
    x = hidden_states.reshape(-1, orig_shape[-1])
    t, hd = x.shape
    ne, di, _ = w1.shape
    ni = di // _BI
    nh = di // _HB
    up_a = pl.BlockSpec((1, _HB, hd), lambda e, i: (e, 2 * i, 0))
    up_b = pl.BlockSpec((1, _HB, hd), lambda e, i: (e, 2 * i + 1, 0))
    dn_a = pl.BlockSpec((1, hd, _HB), lambda e, i: (e, 0, 2 * i))
    dn_b = pl.BlockSpec((1, hd, _HB), lambda e, i: (e, 0, 2 * i + 1))
    out = pl.pallas_call(
        _moe_body,
        grid=(ne, ni),
        compiler_params=pltpu.CompilerParams(
            vmem_limit_bytes=110 * 1024 * 1024),
        in_specs=[
            pl.BlockSpec((t, hd), lambda e, i: (0, 0)),
            pl.BlockSpec((ne, hd), lambda e, i: (0, 0)),
            up_a, up_b, up_a, up_b, dn_a, dn_b,
        ],
        out_specs=pl.BlockSpec((t, hd), lambda e, i: (0, 0)),
        out_shape=jax.ShapeDtypeStruct((t, hd), jnp.float32),
        scratch_shapes=[pltpu.VMEM((t, ne), jnp.float32)],
    )(x, gate_w, w1, w1, w3, w3, w2, w2)
    return out.reshape(orig_shape)


# speedup vs baseline: 1.2613x; 1.0347x over previous
import jax
import jax.numpy as jnp
from jax.experimental import pallas as pl
from jax.experimental.pallas import tpu as pltpu

_NE = 8
_TOPK = 2
_BI = 1024
_HB = _BI // 2


def _router(x, gw):
    logits = jax.lax.dot_general(
        x, gw, (((1,), (1,)), ((), ())), preferred_element_type=jnp.float32)
    logits = 30.0 * jnp.tanh(logits / 30.0)
    m = jnp.max(logits, axis=1, keepdims=True)
    ex = jnp.exp(logits - m)
    p = ex / jnp.sum(ex, axis=1, keepdims=True)
    cols = jax.lax.broadcasted_iota(jnp.int32, (1, _NE), 1)
    rank_cols = []
    for ee in range(_NE):
        pe = p[:, ee:ee + 1]
        gt = jnp.sum((p > pe).astype(jnp.int32), axis=1, keepdims=True)
        eq = jnp.sum(jnp.logical_and(p == pe, cols < ee).astype(jnp.int32),
                     axis=1, keepdims=True)
        rank_cols.append(gt + eq)
    rank = jnp.concatenate(rank_cols, axis=1)
    return jnp.where(rank < _TOPK, p, 0.0)


def _moe_body(x_ref, gw_ref, w1a_ref, w1b_ref, w3a_ref, w3b_ref,
              w2a_ref, w2b_ref, out_ref, cw_ref):
    e = pl.program_id(0)
    i = pl.program_id(1)
    first = jnp.logical_and(e == 0, i == 0)

    @pl.when(first)
    def _():
        cw_ref[...] = _router(x_ref[...], gw_ref[...])

    x = x_ref[...].astype(jnp.bfloat16)
    cd = (((1,), (1,)), ((), ()))
    ha = jax.lax.dot_general(x, w1a_ref[0].astype(jnp.bfloat16), cd,
                             preferred_element_type=jnp.float32)
    ga = jax.lax.dot_general(x, w3a_ref[0].astype(jnp.bfloat16), cd,
                             preferred_element_type=jnp.float32)
    acta = (ha * (1.0 / (1.0 + jnp.exp(-ha))) * ga).astype(jnp.bfloat16)
    hb = jax.lax.dot_general(x, w1b_ref[0].astype(jnp.bfloat16), cd,
                             preferred_element_type=jnp.float32)
    gb = jax.lax.dot_general(x, w3b_ref[0].astype(jnp.bfloat16), cd,
                             preferred_element_type=jnp.float32)
    actb = (hb * (1.0 / (1.0 + jnp.exp(-hb))) * gb).astype(jnp.bfloat16)
    part = (jax.lax.dot_general(acta, w2a_ref[0].astype(jnp.bfloat16), cd,
                                preferred_element_type=jnp.float32)
            + jax.lax.dot_general(actb, w2b_ref[0].astype(jnp.bfloat16), cd,
                                  preferred_element_type=jnp.float32))
    onehot = (jax.lax.broadcasted_iota(jnp.int32, (1, _NE), 1) == e
              ).astype(jnp.float32)
    cw_col = jnp.sum(cw_ref[...] * onehot, axis=1, keepdims=True)
    contrib = cw_col * part

    @pl.when(first)
    def _():
        out_ref[...] = contrib

    @pl.when(jnp.logical_not(first))
    def _():
        out_ref[...] += contrib


def kernel(hidden_states, gate_w, w1, w3, w2):
    orig_shape = hidden_states.shape
    x = hidden_states.reshape(-1, orig_shape[-1])
    t, hd = x.shape
    ne, di, _ = w1.shape
    ni = di // _BI
    nh = di // _HB
    up_a = pl.BlockSpec((1, _HB, hd), lambda e, i: (e, 2 * i, 0))
    up_b = pl.BlockSpec((1, _HB, hd), lambda e, i: (e, 2 * i + 1, 0))
    dn_a = pl.BlockSpec((1, hd, _HB), lambda e, i: (e, 0, 2 * i))
    dn_b = pl.BlockSpec((1, hd, _HB), lambda e, i: (e, 0, 2 * i + 1))
    out = pl.pallas_call(
        _moe_body,
        grid=(ne, ni),
        in_specs=[
            pl.BlockSpec((t, hd), lambda e, i: (0, 0)),
            pl.BlockSpec((ne, hd), lambda e, i: (0, 0)),
            up_a, up_b, up_a, up_b, dn_a, dn_b,
        ],
        out_specs=pl.BlockSpec((t, hd), lambda e, i: (0, 0)),
        out_shape=jax.ShapeDtypeStruct((t, hd), jnp.float32),
        scratch_shapes=[pltpu.VMEM((t, ne), jnp.float32)],
    )(x, gate_w, w1, w1, w3, w3, w2, w2)
    return out.reshape(orig_shape)


# speedup vs baseline: 1.2678x; 1.0051x over previous
import jax
import jax.numpy as jnp
from jax.experimental import pallas as pl
from jax.experimental.pallas import tpu as pltpu

_NE = 8
_TOPK = 2
_BI = 1024
_HB = _BI // 2


def _router(x, gw):
    logits = jax.lax.dot_general(
        x, gw, (((1,), (1,)), ((), ())), preferred_element_type=jnp.float32)
    logits = 30.0 * jnp.tanh(logits / 30.0)
    m = jnp.max(logits, axis=1, keepdims=True)
    ex = jnp.exp(logits - m)
    p = ex / jnp.sum(ex, axis=1, keepdims=True)
    cols = jax.lax.broadcasted_iota(jnp.int32, (1, _NE), 1)
    rank_cols = []
    for ee in range(_NE):
        pe = p[:, ee:ee + 1]
        gt = jnp.sum((p > pe).astype(jnp.int32), axis=1, keepdims=True)
        eq = jnp.sum(jnp.logical_and(p == pe, cols < ee).astype(jnp.int32),
                     axis=1, keepdims=True)
        rank_cols.append(gt + eq)
    rank = jnp.concatenate(rank_cols, axis=1)
    return jnp.where(rank < _TOPK, p, 0.0)


def _moe_body(x_ref, gw_ref, w1a_ref, w1b_ref, w3a_ref, w3b_ref,
              w2a_ref, w2b_ref, out_ref, cw_ref):
    g = pl.program_id(0)
    e = g // 2
    first = g == 0

    @pl.when(first)
    def _():
        cw_ref[...] = _router(x_ref[...], gw_ref[...])

    x = x_ref[...].astype(jnp.bfloat16)
    cd = (((1,), (1,)), ((), ()))
    ha = jax.lax.dot_general(x, w1a_ref[0].astype(jnp.bfloat16), cd,
                             preferred_element_type=jnp.float32)
    ga = jax.lax.dot_general(x, w3a_ref[0].astype(jnp.bfloat16), cd,
                             preferred_element_type=jnp.float32)
    acta = (ha * (1.0 / (1.0 + jnp.exp(-ha))) * ga).astype(jnp.bfloat16)
    hb = jax.lax.dot_general(x, w1b_ref[0].astype(jnp.bfloat16), cd,
                             preferred_element_type=jnp.float32)
    gb = jax.lax.dot_general(x, w3b_ref[0].astype(jnp.bfloat16), cd,
                             preferred_element_type=jnp.float32)
    actb = (hb * (1.0 / (1.0 + jnp.exp(-hb))) * gb).astype(jnp.bfloat16)
    part = (jax.lax.dot_general(acta, w2a_ref[0].astype(jnp.bfloat16), cd,
                                preferred_element_type=jnp.float32)
            + jax.lax.dot_general(actb, w2b_ref[0].astype(jnp.bfloat16), cd,
                                  preferred_element_type=jnp.float32))
    onehot = (jax.lax.broadcasted_iota(jnp.int32, (1, _NE), 1) == e
              ).astype(jnp.float32)
    cw_col = jnp.sum(cw_ref[...] * onehot, axis=1, keepdims=True)
    contrib = cw_col * part

    @pl.when(first)
    def _():
        out_ref[...] = contrib

    @pl.when(jnp.logical_not(first))
    def _():
        out_ref[...] += contrib


def kernel(hidden_states, gate_w, w1, w3, w2):
    orig_shape = hidden_states.shape
    x = hidden_states.reshape(-1, orig_shape[-1])
    t, hd = x.shape
    ne, di, _ = w1.shape
    ni = di // _BI
    nh = di // _HB
    up_a = pl.BlockSpec((1, _HB, hd), lambda g: (g // 2, 2 * (g % 2), 0))
    up_b = pl.BlockSpec((1, _HB, hd), lambda g: (g // 2, 2 * (g % 2) + 1, 0))
    dn_a = pl.BlockSpec((1, hd, _HB), lambda g: (g // 2, 0, 2 * (g % 2)))
    dn_b = pl.BlockSpec((1, hd, _HB), lambda g: (g // 2, 0, 2 * (g % 2) + 1))
    out = pl.pallas_call(
        _moe_body,
        grid=(ne * ni,),
        in_specs=[
            pl.BlockSpec((t, hd), lambda g: (0, 0)),
            pl.BlockSpec((ne, hd), lambda g: (0, 0)),
            up_a, up_b, up_a, up_b, dn_a, dn_b,
        ],
        out_specs=pl.BlockSpec((t, hd), lambda g: (0, 0)),
        out_shape=jax.ShapeDtypeStruct((t, hd), jnp.float32),
        scratch_shapes=[pltpu.VMEM((t, ne), jnp.float32)],
    )(x, gate_w, w1, w1, w3, w3, w2, w2)
    return out.reshape(orig_shape)
